# Initial kernel scaffold; baseline (speedup 1.0000x reference)
import jax
import jax.numpy as jnp
from jax.experimental import pallas as pl


def kernel(node_features, edge_index, edge_features, W1, b1, W2, b2, W3, b3):
    raise NotImplementedError("write your pallas kernel here")


# speedup vs baseline: 3.0412x; 3.0412x over previous
import functools

import jax
import jax.numpy as jnp
from jax import lax
from jax.experimental import pallas as pl
from jax.experimental.pallas import tpu as pltpu
from jax.experimental.pallas import tpu_sc as plsc

NODE_D = 128
HID = 64
EDGE_D = 16

_NC = 2
_NS = 16
_NW = _NC * _NS


def _elu(x):
    return jnp.where(x > 0, x, jnp.exp(x) - 1.0)


def _softplus(x):
    return jnp.maximum(x, 0.0) + jnp.log(1.0 + jnp.exp(-jnp.abs(x)))


def _precompute_body(nf_ref, wst_ref, wrt_ref, ps_ref, pr_ref):
    nf = nf_ref[...]
    ps_ref[...] = jnp.dot(nf, wst_ref[...], preferred_element_type=jnp.float32)
    pr_ref[...] = jnp.dot(nf, wrt_ref[...], preferred_element_type=jnp.float32)


def _mlp_body(g_ref, ef_ref, w1et_ref, b1_ref, w2t_ref, b2_ref, w3_ref, b3_ref,
              out_ref):
    g = g_ref[...]
    ef = ef_ref[...]
    a = jnp.dot(ef, w1et_ref[...], preferred_element_type=jnp.float32)
    h1 = _elu(g + a + b1_ref[...])
    h2 = _elu(jnp.dot(h1, w2t_ref[...], preferred_element_type=jnp.float32)
              + b2_ref[...])
    z = lax.dot_general(w3_ref[...], h2, (((1,), (1,)), ((), ())),
                        preferred_element_type=jnp.float32)
    out_ref[0] = _softplus(z + b3_ref[...])


def _make_gather(n_edges, chunk):
    epw = n_edges // _NW
    nchunk = epw // chunk
    mesh = plsc.VectorSubcoreMesh(core_axis_name="c", subcore_axis_name="s")

    @functools.partial(
        pl.kernel,
        mesh=mesh,
        compiler_params=pltpu.CompilerParams(use_tc_tiling_on_sc=False),
        out_type=jax.ShapeDtypeStruct((n_edges, HID), jnp.float32),
        scratch_types=[
            pltpu.VMEM((chunk,), jnp.int32),
            pltpu.VMEM((chunk,), jnp.int32),
            pltpu.VMEM((chunk, HID), jnp.float32),
            pltpu.VMEM((chunk, HID), jnp.float32),
            pltpu.SemaphoreType.DMA,
            pltpu.SemaphoreType.DMA,
        ],
    )
    def _gather(ps_hbm, pr_hbm, s_hbm, r_hbm, out_hbm, idxs, idxr, bufs, bufr,
                sem_s, sem_r):
        wid = lax.axis_index("s") * _NC + lax.axis_index("c")
        base = wid * epw

        def chunk_body(k, carry):
            off = base + k * chunk
            pltpu.sync_copy(s_hbm.at[pl.ds(off, chunk)], idxs)
            pltpu.sync_copy(r_hbm.at[pl.ds(off, chunk)], idxr)
            cs = pltpu.async_copy(ps_hbm.at[idxs], bufs, sem_s)
            cr = pltpu.async_copy(pr_hbm.at[idxr], bufr, sem_r)
            cs.wait()
            cr.wait()

            def add_row(rr, inner):
                for c4 in range(HID // 16):
                    sl = pl.ds(c4 * 16, 16)
                    bufs[rr, sl] = bufs[rr, sl] + bufr[rr, sl]
                return inner

            lax.fori_loop(0, chunk, add_row, 0)
            pltpu.sync_copy(bufs, out_hbm.at[pl.ds(off, chunk)])
            return carry

        lax.fori_loop(0, nchunk, chunk_body, 0)

    return _gather


def kernel(node_features, edge_index, edge_features, W1, b1, W2, b2, W3, b3):
    n_nodes = node_features.shape[0]
    n_edges = edge_features.shape[0]

    s32 = edge_index[0].astype(jnp.int32)
    r32 = edge_index[1].astype(jnp.int32)
    w1st = W1[:, :NODE_D].T
    w1rt = W1[:, NODE_D:2 * NODE_D].T
    w1et = W1[:, 2 * NODE_D:].T
    b1_2 = b1.reshape(1, HID)
    b2_2 = b2.reshape(1, HID)
    b3_2 = b3.reshape(1, 1)

    ps, pr = pl.pallas_call(
        _precompute_body,
        out_shape=(
            jax.ShapeDtypeStruct((n_nodes, HID), jnp.float32),
            jax.ShapeDtypeStruct((n_nodes, HID), jnp.float32),
        ),
    )(node_features, w1st, w1rt)

    g = _make_gather(n_edges, 400)(ps, pr, s32, r32)

    rows = 3200
    nblocks = n_edges // rows
    out2 = pl.pallas_call(
        _mlp_body,
        grid=(nblocks,),
        in_specs=[
            pl.BlockSpec((rows, HID), lambda i: (i, 0)),
            pl.BlockSpec((rows, EDGE_D), lambda i: (i, 0)),
            pl.BlockSpec((EDGE_D, HID), lambda i: (0, 0)),
            pl.BlockSpec((1, HID), lambda i: (0, 0)),
            pl.BlockSpec((HID, HID), lambda i: (0, 0)),
            pl.BlockSpec((1, HID), lambda i: (0, 0)),
            pl.BlockSpec((1, HID), lambda i: (0, 0)),
            pl.BlockSpec((1, 1), lambda i: (0, 0)),
        ],
        out_specs=pl.BlockSpec((1, 1, rows), lambda i: (i, 0, 0)),
        out_shape=jax.ShapeDtypeStruct((nblocks, 1, rows), jnp.float32),
    )(g, edge_features, w1et, b1_2, W2.T, b2_2, W3, b3_2)
    return out2.reshape(n_edges)
